# baseline (speedup 1.0000x reference)
import functools

import jax
import jax.numpy as jnp
from jax import lax
from jax.experimental import pallas as pl
from jax.experimental.pallas import tpu as pltpu
from jax.experimental.pallas import tpu_sc as plsc

N = 10000
E = 320000
D = 128
C = 40
DP = 48
NC = 2
NS = 16
NW = NC * NS
EPW = E // NW
CH = 128
NCH = 80
EPWP = CH * NCH
NPAD = N + 8
RING = 5
GDEPTH = 4
NPS = N // NS
PCH = 125
BR = 1000
NB = N // BR

_MESH = plsc.VectorSubcoreMesh(core_axis_name="c", subcore_axis_name="s")
_SC_PARAMS = pltpu.CompilerParams(use_tc_tiling_on_sc=False)


def _deg_body(dst_hbm, ones_hbm, zeros_hbm, out_hbm, idx_d, ones_v, acc, sem):
    c = lax.axis_index("c")
    s = lax.axis_index("s")

    pltpu.sync_copy(zeros_hbm.at[pl.ds(s * NPS, NPS)],
                    acc.at[pl.ds(s * NPS, NPS)])
    pltpu.sync_copy(ones_hbm, ones_v)
    wid = s * NC + c
    pltpu.sync_copy(dst_hbm.at[wid], idx_d)
    plsc.subcore_barrier()

    @pl.loop(0, NCH, step=8)
    def _(j):
        for b in range(8):
            pltpu.async_copy(ones_v, acc.at[idx_d.at[j + b]], sem, add=True)
        for b in range(8):
            pltpu.make_async_copy(ones_v, acc.at[idx_d.at[j + b]], sem).wait()

    plsc.subcore_barrier()
    pltpu.sync_copy(acc.at[pl.ds(s * NPS, NPS)],
                    out_hbm.at[c, pl.ds(s * NPS, NPS)])


@functools.partial(
    pl.kernel,
    out_type=jax.ShapeDtypeStruct((NC, N, 16), jnp.float32),
    mesh=_MESH,
    scratch_types=[
        pltpu.VMEM((NCH, CH), jnp.int32),
        pltpu.VMEM((CH, 16), jnp.float32),
        pltpu.VMEM_SHARED((NPAD, 16), jnp.float32),
        pltpu.SemaphoreType.DMA,
    ],
    compiler_params=_SC_PARAMS,
)
def _deg_kernel(dst_hbm, ones_hbm, zeros_hbm, out_hbm, idx_d, ones_v, acc,
                sem):
    _deg_body(dst_hbm, ones_hbm, zeros_hbm, out_hbm, idx_d, ones_v, acc, sem)


def _hop_pipeline(src_hbm, dst_hbm, zeros_hbm, out_hbm,
                  idx_s, idx_d, rows, acc, gs, gsems, ssems, c, s):
    wid = s * NC + c
    pltpu.sync_copy(src_hbm.at[wid], idx_s)
    pltpu.sync_copy(dst_hbm.at[wid], idx_d)
    pltpu.sync_copy(zeros_hbm.at[pl.ds(s * NPS, NPS)],
                    acc.at[pl.ds(s * NPS, NPS)])
    plsc.subcore_barrier()

    for m in range(GDEPTH):
        pltpu.async_copy(gs.at[idx_s.at[m]], rows[m], gsems[m])

    @pl.loop(0, NCH, step=RING)
    def _(j):
        for r in range(RING):
            k = j + r
            b2 = (r + GDEPTH) % RING

            @pl.when((k >= RING - GDEPTH) & (k + GDEPTH < NCH))
            def _():
                pltpu.make_async_copy(rows[b2], acc.at[idx_d.at[k]],
                                      ssems[b2]).wait()

            @pl.when(k + GDEPTH < NCH)
            def _():
                pltpu.async_copy(gs.at[idx_s.at[k + GDEPTH]], rows[b2],
                                 gsems[b2])

            pltpu.make_async_copy(gs.at[idx_s.at[k]], rows[r],
                                  gsems[r]).wait()
            pltpu.async_copy(rows[r], acc.at[idx_d.at[k]], ssems[r],
                             add=True)

    for r in range(RING):
        pltpu.make_async_copy(rows[r], acc.at[idx_d.at[r]], ssems[r]).wait()

    plsc.subcore_barrier()
    pltpu.sync_copy(acc.at[pl.ds(s * NPS, NPS)],
                    out_hbm.at[c, pl.ds(s * NPS, NPS)])


def _hop1_body(g_hbm, src_hbm, dst_hbm, zeros_hbm, out_hbm,
               idx_s, idx_d, rows, acc, gs, gsems, ssems):
    c = lax.axis_index("c")
    s = lax.axis_index("s")
    pltpu.sync_copy(g_hbm.at[pl.ds(s * NPS, NPS)], gs.at[pl.ds(s * NPS, NPS)])
    _hop_pipeline(src_hbm, dst_hbm, zeros_hbm, out_hbm,
                  idx_s, idx_d, rows, acc, gs, gsems, ssems, c, s)


def _hop2_body(sp_hbm, g1_hbm, r2_hbm, src_hbm, dst_hbm, zeros_hbm, out_hbm,
               idx_s, idx_d, rows, acc, gs, r2v, gsems, ssems):
    c = lax.axis_index("c")
    s = lax.axis_index("s")
    p0v, p1v, g1v = rows[0], rows[1], rows[2]
    @pl.loop(0, NPS, step=PCH)
    def _(q):
        r0 = s * NPS + q
        pltpu.sync_copy(sp_hbm.at[0, pl.ds(r0, PCH)], p0v.at[pl.ds(0, PCH)])
        pltpu.sync_copy(sp_hbm.at[1, pl.ds(r0, PCH)], p1v.at[pl.ds(0, PCH)])
        pltpu.sync_copy(g1_hbm.at[pl.ds(r0, PCH)], g1v.at[pl.ds(0, PCH)])
        pltpu.sync_copy(r2_hbm.at[pl.ds(r0, PCH)], r2v)

        @pl.loop(0, PCH)
        def _(r):
            rr = r2v[r, pl.ds(0, 16)]
            for cc in range(DP // 16):
                sl = pl.ds(16 * cc, 16)
                g1v[r, sl] = (p0v[r, sl] + p1v[r, sl] + g1v[r, sl]) * rr

        pltpu.sync_copy(g1v.at[pl.ds(0, PCH)], gs.at[pl.ds(r0, PCH)])

    _hop_pipeline(src_hbm, dst_hbm, zeros_hbm, out_hbm,
                  idx_s, idx_d, rows, acc, gs, gsems, ssems, c, s)


_HOP_SCRATCH = [
    pltpu.VMEM((NCH, CH), jnp.int32),
    pltpu.VMEM((NCH, CH), jnp.int32),
    [pltpu.VMEM((CH, DP), jnp.float32)] * RING,
    pltpu.VMEM_SHARED((NPAD, DP), jnp.float32),
    pltpu.VMEM_SHARED((N, DP), jnp.float32),
]
_HOP_SEMS = [
    [pltpu.SemaphoreType.DMA] * RING,
    [pltpu.SemaphoreType.DMA] * RING,
]


@functools.partial(
    pl.kernel,
    out_type=jax.ShapeDtypeStruct((NC, N, DP), jnp.float32),
    mesh=_MESH,
    scratch_types=_HOP_SCRATCH + _HOP_SEMS,
    compiler_params=_SC_PARAMS,
)
def _hop1_kernel(g_hbm, src_hbm, dst_hbm, zeros_hbm, out_hbm,
                 idx_s, idx_d, rows, acc, gs, gsems, ssems):
    _hop1_body(g_hbm, src_hbm, dst_hbm, zeros_hbm, out_hbm,
               idx_s, idx_d, rows, acc, gs, gsems, ssems)


@functools.partial(
    pl.kernel,
    out_type=jax.ShapeDtypeStruct((NC, N, DP), jnp.float32),
    mesh=_MESH,
    scratch_types=_HOP_SCRATCH + [
        pltpu.VMEM((PCH, 16), jnp.float32),
    ] + _HOP_SEMS,
    compiler_params=_SC_PARAMS,
)
def _hop2_kernel(sp_hbm, g1_hbm, r2_hbm, src_hbm, dst_hbm, zeros_hbm, out_hbm,
                 idx_s, idx_d, rows, acc, gs, r2v, gsems, ssems):
    _hop2_body(sp_hbm, g1_hbm, r2_hbm, src_hbm, dst_hbm, zeros_hbm, out_hbm,
               idx_s, idx_d, rows, acc, gs, r2v, gsems, ssems)


def _prep_body(x_ref, w_ref, d0_ref, d1_ref, g_ref, r2_ref):
    deg = 1.0 + d0_ref[0, :, 0:1] + d1_ref[0, :, 0:1]
    z = lax.dot_general(x_ref[...], w_ref[...], (((1,), (1,)), ((), ())),
                        preferred_element_type=jnp.float32)
    g_ref[...] = z * lax.rsqrt(deg)
    r2_ref[...] = jnp.broadcast_to(1.0 / deg, (BR, 16))


def _prep(x, wp, degp):
    return pl.pallas_call(
        _prep_body,
        grid=(NB,),
        in_specs=[
            pl.BlockSpec((BR, D), lambda i: (i, 0)),
            pl.BlockSpec((DP, D), lambda i: (0, 0)),
            pl.BlockSpec((1, BR, 16), lambda i: (0, i, 0)),
            pl.BlockSpec((1, BR, 16), lambda i: (1, i, 0)),
        ],
        out_specs=[
            pl.BlockSpec((BR, DP), lambda i: (i, 0)),
            pl.BlockSpec((BR, 16), lambda i: (i, 0)),
        ],
        out_shape=[
            jax.ShapeDtypeStruct((N, DP), jnp.float32),
            jax.ShapeDtypeStruct((N, 16), jnp.float32),
        ],
    )(x, wp, degp, degp)


def _final_body(s1p0, s1p1, s2p0, s2p1, g1_ref, r2_ref, d0_ref, d1_ref,
                b_ref, o_ref):
    deg = 1.0 + d0_ref[0, :, 0:1] + d1_ref[0, :, 0:1]
    g2 = (s1p0[0] + s1p1[0] + g1_ref[...]) * r2_ref[:, 0:1]
    logits = (s2p0[0] + s2p1[0] + g2) * lax.rsqrt(deg) + b_ref[...]
    col = lax.broadcasted_iota(jnp.int32, (BR, DP), 1)
    valid = col < C
    masked = jnp.where(valid, logits, -1e30)
    m = jnp.max(masked, axis=1, keepdims=True)
    e = jnp.where(valid, jnp.exp(logits - m), 0.0)
    lse = jnp.log(jnp.sum(e, axis=1, keepdims=True))
    o_ref[...] = logits - m - lse


def _final(s1p, s2p, g1, r2, degp, bp):
    return pl.pallas_call(
        _final_body,
        grid=(NB,),
        in_specs=[
            pl.BlockSpec((1, BR, DP), lambda i: (0, i, 0)),
            pl.BlockSpec((1, BR, DP), lambda i: (1, i, 0)),
            pl.BlockSpec((1, BR, DP), lambda i: (0, i, 0)),
            pl.BlockSpec((1, BR, DP), lambda i: (1, i, 0)),
            pl.BlockSpec((BR, DP), lambda i: (i, 0)),
            pl.BlockSpec((BR, 16), lambda i: (i, 0)),
            pl.BlockSpec((1, BR, 16), lambda i: (0, i, 0)),
            pl.BlockSpec((1, BR, 16), lambda i: (1, i, 0)),
            pl.BlockSpec((1, DP), lambda i: (0, 0)),
        ],
        out_specs=pl.BlockSpec((BR, DP), lambda i: (i, 0)),
        out_shape=jax.ShapeDtypeStruct((N, DP), jnp.float32),
    )(s1p, s1p, s2p, s2p, g1, r2, degp, degp, bp)


def kernel(x, edge_index, W, b):
    src = edge_index[0].astype(jnp.int32)
    dst = edge_index[1].astype(jnp.int32)
    pad = jnp.zeros((NW, EPWP - EPW), jnp.int32)
    src3 = jnp.concatenate([src.reshape(NW, EPW), pad], axis=1)
    src3 = src3.reshape(NW, NCH, CH)
    dst3 = jnp.concatenate([dst.reshape(NW, EPW), pad + N], axis=1)
    dst3 = dst3.reshape(NW, NCH, CH)

    wp = jnp.zeros((DP, D), jnp.float32).at[:C].set(W)
    bp = jnp.zeros((1, DP), jnp.float32).at[0, :C].set(b)
    ones16 = jnp.ones((CH, 16), jnp.float32)
    zeros16 = jnp.zeros((N, 16), jnp.float32)
    zerosdp = jnp.zeros((N, DP), jnp.float32)

    degp = _deg_kernel(dst3, ones16, zeros16)
    g1, r2 = _prep(x, wp, degp)
    s1p = _hop1_kernel(g1, src3, dst3, zerosdp)
    s2p = _hop2_kernel(s1p, g1, r2, src3, dst3, zerosdp)
    out = _final(s1p, s2p, g1, r2, degp, bp)
    return out[:, :C]


# speedup vs baseline: 1.0587x; 1.0587x over previous
import functools

import jax
import jax.numpy as jnp
from jax import lax
from jax.experimental import pallas as pl
from jax.experimental.pallas import tpu as pltpu
from jax.experimental.pallas import tpu_sc as plsc

N = 10000
E = 320000
D = 128
C = 40
DP = 48
NC = 2
NS = 16
NW = NC * NS
EPW = E // NW
CH = 80
NCH = EPW // CH
NPAD = N
RING = 5
GDEPTH = 4
NPS = N // NS
PCH = 125
BR = 1000
NB = N // BR

_MESH = plsc.VectorSubcoreMesh(core_axis_name="c", subcore_axis_name="s")
_SC_PARAMS = pltpu.CompilerParams(use_tc_tiling_on_sc=False)


def _deg_body(dst_hbm, ones_hbm, zeros_hbm, out_hbm, idx_d, ones_v, acc, sem):
    c = lax.axis_index("c")
    s = lax.axis_index("s")

    pltpu.sync_copy(zeros_hbm.at[pl.ds(s * NPS, NPS)],
                    acc.at[pl.ds(s * NPS, NPS)])
    pltpu.sync_copy(ones_hbm, ones_v)
    wid = s * NC + c
    pltpu.sync_copy(dst_hbm.at[wid], idx_d)
    plsc.subcore_barrier()

    @pl.loop(0, NCH, step=5)
    def _(j):
        for b in range(5):
            pltpu.async_copy(ones_v, acc.at[idx_d.at[j + b]], sem, add=True)
        for b in range(5):
            pltpu.make_async_copy(ones_v, acc.at[idx_d.at[j + b]], sem).wait()

    plsc.subcore_barrier()
    pltpu.sync_copy(acc.at[pl.ds(s * NPS, NPS)],
                    out_hbm.at[c, pl.ds(s * NPS, NPS)])


@functools.partial(
    pl.kernel,
    out_type=jax.ShapeDtypeStruct((NC, N, 16), jnp.float32),
    mesh=_MESH,
    scratch_types=[
        pltpu.VMEM((NCH, CH), jnp.int32),
        pltpu.VMEM((CH, 16), jnp.float32),
        pltpu.VMEM_SHARED((NPAD, 16), jnp.float32),
        pltpu.SemaphoreType.DMA,
    ],
    compiler_params=_SC_PARAMS,
)
def _deg_kernel(dst_hbm, ones_hbm, zeros_hbm, out_hbm, idx_d, ones_v, acc,
                sem):
    _deg_body(dst_hbm, ones_hbm, zeros_hbm, out_hbm, idx_d, ones_v, acc, sem)


def _hop_pipeline(src_hbm, dst_hbm, zeros_hbm, out_hbm,
                  idx_s, idx_d, rows, acc, gs, gsems, ssems, c, s):
    wid = s * NC + c
    pltpu.sync_copy(src_hbm.at[wid], idx_s)
    pltpu.sync_copy(dst_hbm.at[wid], idx_d)
    pltpu.sync_copy(zeros_hbm.at[pl.ds(s * NPS, NPS)],
                    acc.at[pl.ds(s * NPS, NPS)])
    plsc.subcore_barrier()

    for m in range(GDEPTH):
        pltpu.async_copy(gs.at[idx_s.at[m]], rows[m], gsems[m])

    @pl.loop(0, NCH, step=RING)
    def _(j):
        for r in range(RING):
            k = j + r
            b2 = (r + GDEPTH) % RING

            @pl.when((k >= RING - GDEPTH) & (k + GDEPTH < NCH))
            def _():
                pltpu.make_async_copy(rows[b2], acc.at[idx_d.at[k]],
                                      ssems[b2]).wait()

            @pl.when(k + GDEPTH < NCH)
            def _():
                pltpu.async_copy(gs.at[idx_s.at[k + GDEPTH]], rows[b2],
                                 gsems[b2])

            pltpu.make_async_copy(gs.at[idx_s.at[k]], rows[r],
                                  gsems[r]).wait()
            pltpu.async_copy(rows[r], acc.at[idx_d.at[k]], ssems[r],
                             add=True)

    for r in range(RING):
        pltpu.make_async_copy(rows[r], acc.at[idx_d.at[r]], ssems[r]).wait()

    plsc.subcore_barrier()
    pltpu.sync_copy(acc.at[pl.ds(s * NPS, NPS)],
                    out_hbm.at[c, pl.ds(s * NPS, NPS)])


def _hop1_body(g_hbm, src_hbm, dst_hbm, zeros_hbm, out_hbm,
               idx_s, idx_d, rows, acc, gs, gsems, ssems):
    c = lax.axis_index("c")
    s = lax.axis_index("s")
    pltpu.sync_copy(g_hbm.at[pl.ds(s * NPS, NPS)], gs.at[pl.ds(s * NPS, NPS)])
    _hop_pipeline(src_hbm, dst_hbm, zeros_hbm, out_hbm,
                  idx_s, idx_d, rows, acc, gs, gsems, ssems, c, s)


def _hop2_body(sp_hbm, g1_hbm, r2_hbm, src_hbm, dst_hbm, zeros_hbm, out_hbm,
               g2_hbm, idx_s, idx_d, rows, acc, gs, p0v, p1v, g1v, r2v,
               gsems, ssems):
    c = lax.axis_index("c")
    s = lax.axis_index("s")
    @pl.loop(0, NPS, step=PCH)
    def _(q):
        r0 = s * NPS + q
        pltpu.sync_copy(sp_hbm.at[0, pl.ds(r0, PCH)], p0v)
        pltpu.sync_copy(sp_hbm.at[1, pl.ds(r0, PCH)], p1v)
        pltpu.sync_copy(g1_hbm.at[pl.ds(r0, PCH)], g1v)
        pltpu.sync_copy(r2_hbm.at[pl.ds(r0, PCH)], r2v)

        @pl.loop(0, PCH)
        def _(r):
            rr = r2v[r, pl.ds(0, 16)]
            for cc in range(DP // 16):
                sl = pl.ds(16 * cc, 16)
                g1v[r, sl] = (p0v[r, sl] + p1v[r, sl] + g1v[r, sl]) * rr

        pltpu.sync_copy(g1v, gs.at[pl.ds(r0, PCH)])

        @pl.when(c == 0)
        def _():
            pltpu.sync_copy(g1v, g2_hbm.at[pl.ds(r0, PCH)])

    _hop_pipeline(src_hbm, dst_hbm, zeros_hbm, out_hbm,
                  idx_s, idx_d, rows, acc, gs, gsems, ssems, c, s)


_HOP_SCRATCH = [
    pltpu.VMEM((NCH, CH), jnp.int32),
    pltpu.VMEM((NCH, CH), jnp.int32),
    [pltpu.VMEM((CH, DP), jnp.float32)] * RING,
    pltpu.VMEM_SHARED((NPAD, DP), jnp.float32),
    pltpu.VMEM_SHARED((N, DP), jnp.float32),
]
_HOP_SEMS = [
    [pltpu.SemaphoreType.DMA] * RING,
    [pltpu.SemaphoreType.DMA] * RING,
]


@functools.partial(
    pl.kernel,
    out_type=jax.ShapeDtypeStruct((NC, N, DP), jnp.float32),
    mesh=_MESH,
    scratch_types=_HOP_SCRATCH + _HOP_SEMS,
    compiler_params=_SC_PARAMS,
)
def _hop1_kernel(g_hbm, src_hbm, dst_hbm, zeros_hbm, out_hbm,
                 idx_s, idx_d, rows, acc, gs, gsems, ssems):
    _hop1_body(g_hbm, src_hbm, dst_hbm, zeros_hbm, out_hbm,
               idx_s, idx_d, rows, acc, gs, gsems, ssems)


@functools.partial(
    pl.kernel,
    out_type=[jax.ShapeDtypeStruct((NC, N, DP), jnp.float32),
              jax.ShapeDtypeStruct((N, DP), jnp.float32)],
    mesh=_MESH,
    scratch_types=_HOP_SCRATCH + [
        pltpu.VMEM((PCH, DP), jnp.float32),
        pltpu.VMEM((PCH, DP), jnp.float32),
        pltpu.VMEM((PCH, DP), jnp.float32),
        pltpu.VMEM((PCH, 16), jnp.float32),
    ] + _HOP_SEMS,
    compiler_params=_SC_PARAMS,
)
def _hop2_kernel(sp_hbm, g1_hbm, r2_hbm, src_hbm, dst_hbm, zeros_hbm, out_hbm,
                 g2_hbm, idx_s, idx_d, rows, acc, gs, p0v, p1v, g1v, r2v,
                 gsems, ssems):
    _hop2_body(sp_hbm, g1_hbm, r2_hbm, src_hbm, dst_hbm, zeros_hbm, out_hbm,
               g2_hbm, idx_s, idx_d, rows, acc, gs, p0v, p1v, g1v, r2v,
               gsems, ssems)


def _prep_body(x_ref, w_ref, d0_ref, d1_ref, g_ref, r2_ref):
    deg = 1.0 + d0_ref[0, :, 0:1] + d1_ref[0, :, 0:1]
    z = lax.dot_general(x_ref[...], w_ref[...], (((1,), (1,)), ((), ())),
                        preferred_element_type=jnp.float32)
    g_ref[...] = z * lax.rsqrt(deg)
    r2_ref[...] = jnp.broadcast_to(1.0 / deg, (BR, 16))


def _prep(x, wp, degp):
    return pl.pallas_call(
        _prep_body,
        grid=(NB,),
        in_specs=[
            pl.BlockSpec((BR, D), lambda i: (i, 0)),
            pl.BlockSpec((DP, D), lambda i: (0, 0)),
            pl.BlockSpec((1, BR, 16), lambda i: (0, i, 0)),
            pl.BlockSpec((1, BR, 16), lambda i: (1, i, 0)),
        ],
        out_specs=[
            pl.BlockSpec((BR, DP), lambda i: (i, 0)),
            pl.BlockSpec((BR, 16), lambda i: (i, 0)),
        ],
        out_shape=[
            jax.ShapeDtypeStruct((N, DP), jnp.float32),
            jax.ShapeDtypeStruct((N, 16), jnp.float32),
        ],
    )(x, wp, degp, degp)


def _final_body(s2p0, s2p1, g2_ref, d0_ref, d1_ref, b_ref, o_ref):
    deg = 1.0 + d0_ref[0, :, 0:1] + d1_ref[0, :, 0:1]
    logits = (s2p0[0] + s2p1[0] + g2_ref[...]) * lax.rsqrt(deg) + b_ref[...]
    col = lax.broadcasted_iota(jnp.int32, (BR, DP), 1)
    valid = col < C
    masked = jnp.where(valid, logits, -1e30)
    m = jnp.max(masked, axis=1, keepdims=True)
    e = jnp.where(valid, jnp.exp(logits - m), 0.0)
    lse = jnp.log(jnp.sum(e, axis=1, keepdims=True))
    o_ref[...] = (logits - m - lse)[:, :C]


def _final(s2p, g2, degp, bp):
    return pl.pallas_call(
        _final_body,
        grid=(NB,),
        in_specs=[
            pl.BlockSpec((1, BR, DP), lambda i: (0, i, 0)),
            pl.BlockSpec((1, BR, DP), lambda i: (1, i, 0)),
            pl.BlockSpec((BR, DP), lambda i: (i, 0)),
            pl.BlockSpec((1, BR, 16), lambda i: (0, i, 0)),
            pl.BlockSpec((1, BR, 16), lambda i: (1, i, 0)),
            pl.BlockSpec((1, DP), lambda i: (0, 0)),
        ],
        out_specs=pl.BlockSpec((BR, C), lambda i: (i, 0)),
        out_shape=jax.ShapeDtypeStruct((N, C), jnp.float32),
    )(s2p, s2p, g2, degp, degp, bp)


def kernel(x, edge_index, W, b):
    src = edge_index[0].astype(jnp.int32)
    dst = edge_index[1].astype(jnp.int32)
    src3 = src.reshape(NW, NCH, CH)
    dst3 = dst.reshape(NW, NCH, CH)

    wp = jnp.zeros((DP, D), jnp.float32).at[:C].set(W)
    bp = jnp.zeros((1, DP), jnp.float32).at[0, :C].set(b)
    ones16 = jnp.ones((CH, 16), jnp.float32)
    zeros16 = jnp.zeros((N, 16), jnp.float32)
    zerosdp = jnp.zeros((N, DP), jnp.float32)

    degp = _deg_kernel(dst3, ones16, zeros16)
    g1, r2 = _prep(x, wp, degp)
    s1p = _hop1_kernel(g1, src3, dst3, zerosdp)
    s2p, g2 = _hop2_kernel(s1p, g1, r2, src3, dst3, zerosdp)
    return _final(s2p, g2, degp, bp)
